# baseline (device time: 47622 ns/iter reference)
import jax
import jax.numpy as jnp
from jax import lax
from jax.experimental import pallas as pl
from jax.experimental.pallas import tpu as pltpu

N_DEV = 4
WIN = 128
QB = 128
KB = 3 * WIN


def kernel(x, Wq, K_ext, V_ext, Wo):
    B, Sq, HD = x.shape
    _, Skv, Hq, Dh = K_ext.shape
    Dm = Wq.shape[1]
    Sh = Skv + 2 * WIN
    NQB = Sq // QB

    def body(x_ref, wq_ref, k_ref, v_ref, wo_ref, out_ref,
             kbuf, vbuf, send_sems, recv_sems):
        my = lax.axis_index("i")
        left = lax.rem(my + N_DEV - 1, N_DEV)
        right = lax.rem(my + 1, N_DEV)

        barrier_sem = pltpu.get_barrier_semaphore()
        for nbr in (left, right):
            pl.semaphore_signal(
                barrier_sem, inc=1,
                device_id=(nbr,), device_id_type=pl.DeviceIdType.MESH,
            )
        pl.semaphore_wait(barrier_sem, 2)

        to_left, to_right = [], []
        for idx, (src, dbuf) in enumerate([(k_ref, kbuf), (v_ref, vbuf)]):
            r = pltpu.make_async_remote_copy(
                src_ref=src.at[:, pl.ds(0, WIN)],
                dst_ref=dbuf.at[:, pl.ds(WIN + Skv, WIN)],
                send_sem=send_sems.at[idx],
                recv_sem=recv_sems.at[idx],
                device_id=(left,), device_id_type=pl.DeviceIdType.MESH,
            )
            r.start()
            to_left.append(r)
        for idx, (src, dbuf) in enumerate([(k_ref, kbuf), (v_ref, vbuf)], 2):
            r = pltpu.make_async_remote_copy(
                src_ref=src.at[:, pl.ds(Skv - WIN, WIN)],
                dst_ref=dbuf.at[:, pl.ds(0, WIN)],
                send_sem=send_sems.at[idx],
                recv_sem=recv_sems.at[idx],
                device_id=(right,), device_id_type=pl.DeviceIdType.MESH,
            )
            r.start()
            to_right.append(r)

        kbuf[:, WIN:WIN + Skv] = k_ref[...]
        vbuf[:, WIN:WIN + Skv] = v_ref[...]

        xr = x_ref[...].reshape(B * Sq, HD).astype(jnp.bfloat16)
        Q = lax.dot(xr, wq_ref[...].astype(jnp.bfloat16),
                    preferred_element_type=jnp.float32)
        Qbf = Q.astype(jnp.bfloat16)

        qi = lax.broadcasted_iota(jnp.int32, (Sq, Sh), 0)
        ki = lax.broadcasted_iota(jnp.int32, (Sq, Sh), 1)
        ki_g = my * Skv - WIN + ki
        valid = (jnp.abs(qi - ki + WIN) <= WIN) & (ki_g >= 0) & (ki_g < N_DEV * Skv)
        neg = jnp.float32(-1e9)

        def attn_block(b, h, qb, kb_val, vb_val):
            q = Qbf[b * Sq + qb * QB:b * Sq + (qb + 1) * QB,
                    h * Dh:(h + 1) * Dh]
            k = kb_val[b, qb * QB:qb * QB + KB, h, :].astype(jnp.bfloat16)
            s = lax.dot_general(
                q, k, (((1,), (1,)), ((), ())),
                preferred_element_type=jnp.float32,
            ) * 0.125
            vmask = valid[qb * QB:(qb + 1) * QB, qb * QB:qb * QB + KB]
            p = jnp.exp(jnp.where(vmask, s, neg))
            denom = jnp.sum(p, axis=-1, keepdims=True)
            v = vb_val[b, qb * QB:qb * QB + KB, h, :].astype(jnp.bfloat16)
            return lax.dot(p.astype(jnp.bfloat16), v,
                           preferred_element_type=jnp.float32) / denom

        ctx = {}

        kb_val = kbuf[...]
        vb_val = vbuf[...]
        for b in range(B):
            for h in range(Hq):
                for qb in (1, 2):
                    ctx[b, h, qb] = attn_block(b, h, qb, kb_val, vb_val)

        for r in to_right:
            r.wait_recv()
        kb_val = kbuf[...]
        vb_val = vbuf[...]
        for b in range(B):
            for h in range(Hq):
                ctx[b, h, 0] = attn_block(b, h, 0, kb_val, vb_val)

        for r in to_left:
            r.wait_recv()
        kb_val = kbuf[...]
        vb_val = vbuf[...]
        for b in range(B):
            for h in range(Hq):
                ctx[b, h, NQB - 1] = attn_block(b, h, NQB - 1, kb_val, vb_val)

        for r in to_left + to_right:
            r.wait_send()

        for b in range(B):
            cb = jnp.concatenate(
                [jnp.concatenate([ctx[b, h, qb] for qb in range(NQB)], axis=0)
                 for h in range(Hq)], axis=1)
            out_ref[b] = lax.dot(cb.astype(jnp.bfloat16),
                                 wo_ref[...].astype(jnp.bfloat16),
                                 preferred_element_type=jnp.float32)

    return pl.pallas_call(
        body,
        out_shape=jax.ShapeDtypeStruct((B, Sq, HD), jnp.float32),
        in_specs=[pl.BlockSpec(memory_space=pltpu.VMEM)] * 5,
        out_specs=pl.BlockSpec(memory_space=pltpu.VMEM),
        scratch_shapes=[
            pltpu.VMEM((B, Sh, Hq, Dh), jnp.float32),
            pltpu.VMEM((B, Sh, Hq, Dh), jnp.float32),
            pltpu.SemaphoreType.DMA((4,)),
            pltpu.SemaphoreType.DMA((4,)),
        ],
        compiler_params=pltpu.CompilerParams(collective_id=0),
    )(x, Wq, K_ext, V_ext, Wo)


# device time: 42948 ns/iter; 1.1088x vs baseline; 1.1088x over previous
import jax
import jax.numpy as jnp
from jax import lax
from jax.experimental import pallas as pl
from jax.experimental.pallas import tpu as pltpu

N_DEV = 4
WIN = 128
QB = 128
KB = 3 * WIN


def kernel(x, Wq, K_ext, V_ext, Wo):
    B, Sq, HD = x.shape
    _, Skv, Hq, Dh = K_ext.shape
    Dm = Wq.shape[1]
    NQB = Sq // QB

    def body(x_ref, wq_ref, k_ref, v_ref, wo_ref, out_ref,
             lhalo_k, lhalo_v, rhalo_k, rhalo_v,
             stage_lo_k, stage_lo_v, stage_hi_k, stage_hi_v,
             send_sems, recv_sems):
        my = lax.axis_index("i")
        left = lax.rem(my + N_DEV - 1, N_DEV)
        right = lax.rem(my + 1, N_DEV)

        barrier_sem = pltpu.get_barrier_semaphore()
        for nbr in (left, right):
            pl.semaphore_signal(
                barrier_sem, inc=1,
                device_id=(nbr,), device_id_type=pl.DeviceIdType.MESH,
            )
        pl.semaphore_wait(barrier_sem, 2)

        stage_lo_k[...] = k_ref[:, :WIN]
        stage_lo_v[...] = v_ref[:, :WIN]
        stage_hi_k[...] = k_ref[:, Skv - WIN:]
        stage_hi_v[...] = v_ref[:, Skv - WIN:]

        to_left, to_right = [], []
        for idx, (src, dst) in enumerate(
                [(stage_lo_k, rhalo_k), (stage_lo_v, rhalo_v)]):
            r = pltpu.make_async_remote_copy(
                src_ref=src, dst_ref=dst,
                send_sem=send_sems.at[idx],
                recv_sem=recv_sems.at[idx],
                device_id=(left,), device_id_type=pl.DeviceIdType.MESH,
            )
            r.start()
            to_left.append(r)
        for idx, (src, dst) in enumerate(
                [(stage_hi_k, lhalo_k), (stage_hi_v, lhalo_v)], 2):
            r = pltpu.make_async_remote_copy(
                src_ref=src, dst_ref=dst,
                send_sem=send_sems.at[idx],
                recv_sem=recv_sems.at[idx],
                device_id=(right,), device_id_type=pl.DeviceIdType.MESH,
            )
            r.start()
            to_right.append(r)

        xr = x_ref[...].reshape(B * Sq, HD).astype(jnp.bfloat16)
        Q = lax.dot(xr, wq_ref[...].astype(jnp.bfloat16),
                    preferred_element_type=jnp.float32)
        Qbf = Q.astype(jnp.bfloat16)

        bi = lax.broadcasted_iota(jnp.int32, (QB, KB), 0)
        bj = lax.broadcasted_iota(jnp.int32, (QB, KB), 1)
        band = jnp.abs(bi - bj + WIN) <= WIN
        neg = jnp.float32(-1e9)

        kv_local = [(k_ref[b][...].astype(jnp.bfloat16),
                     v_ref[b][...].astype(jnp.bfloat16)) for b in range(B)]

        def attn_block(b, h, qb, halo):
            kl, vl = kv_local[b]
            lo = qb * QB - WIN
            if lo < 0:
                hk, hv = halo
                k = jnp.concatenate([hk[b, :, h, :], kl[:KB + lo, h, :]], 0)
                v = jnp.concatenate([hv[b, :, h, :], vl[:KB + lo, h, :]], 0)
                vmask = band & ((bj >= WIN) | (my > 0))
            elif lo + KB > Skv:
                hk, hv = halo
                k = jnp.concatenate([kl[lo:, h, :], hk[b, :, h, :]], 0)
                v = jnp.concatenate([vl[lo:, h, :], hv[b, :, h, :]], 0)
                vmask = band & ((bj < KB - WIN) | (my < N_DEV - 1))
            else:
                k = kl[lo:lo + KB, h, :]
                v = vl[lo:lo + KB, h, :]
                vmask = band
            q = Qbf[b * Sq + qb * QB:b * Sq + (qb + 1) * QB,
                    h * Dh:(h + 1) * Dh]
            s = lax.dot_general(
                q, k, (((1,), (1,)), ((), ())),
                preferred_element_type=jnp.float32,
            ) * 0.125
            p = jnp.exp(jnp.where(vmask, s, neg))
            denom = jnp.sum(p, axis=-1, keepdims=True)
            return lax.dot(p.astype(jnp.bfloat16), v,
                           preferred_element_type=jnp.float32) / denom

        ctx = {}

        for b in range(B):
            for h in range(Hq):
                for qb in (1, 2):
                    ctx[b, h, qb] = attn_block(b, h, qb, None)

        for r in to_right:
            r.wait_recv()
        lk = lhalo_k[...].astype(jnp.bfloat16)
        lv = lhalo_v[...].astype(jnp.bfloat16)
        for b in range(B):
            for h in range(Hq):
                ctx[b, h, 0] = attn_block(b, h, 0, (lk, lv))

        for r in to_left:
            r.wait_recv()
        rk = rhalo_k[...].astype(jnp.bfloat16)
        rv = rhalo_v[...].astype(jnp.bfloat16)
        for b in range(B):
            for h in range(Hq):
                ctx[b, h, NQB - 1] = attn_block(b, h, NQB - 1, (rk, rv))

        for r in to_left + to_right:
            r.wait_send()

        wo_bf = wo_ref[...].astype(jnp.bfloat16)
        for b in range(B):
            cb = jnp.concatenate(
                [jnp.concatenate([ctx[b, h, qb] for qb in range(NQB)], axis=0)
                 for h in range(Hq)], axis=1)
            out_ref[b] = lax.dot(cb.astype(jnp.bfloat16), wo_bf,
                                 preferred_element_type=jnp.float32)

    halo = pltpu.VMEM((B, WIN, Hq, Dh), jnp.float32)
    return pl.pallas_call(
        body,
        out_shape=jax.ShapeDtypeStruct((B, Sq, HD), jnp.float32),
        in_specs=[pl.BlockSpec(memory_space=pltpu.VMEM)] * 5,
        out_specs=pl.BlockSpec(memory_space=pltpu.VMEM),
        scratch_shapes=[
            halo, halo, halo, halo,
            halo, halo, halo, halo,
            pltpu.SemaphoreType.DMA((4,)),
            pltpu.SemaphoreType.DMA((4,)),
        ],
        compiler_params=pltpu.CompilerParams(collective_id=0),
    )(x, Wq, K_ext, V_ext, Wo)


# device time: 27366 ns/iter; 1.7402x vs baseline; 1.5694x over previous
import jax
import jax.numpy as jnp
from jax import lax
from jax.experimental import pallas as pl
from jax.experimental.pallas import tpu as pltpu

N_DEV = 4
WIN = 128
QB = 128
KB = 3 * WIN


def kernel(x, Wq, K_ext, V_ext, Wo):
    B, Sq, HD = x.shape
    _, Skv, Hq, Dh = K_ext.shape
    NQB = Sq // QB

    Kt = jnp.transpose(K_ext, (0, 2, 3, 1))
    Vt = jnp.transpose(V_ext, (0, 2, 3, 1))

    def body(x_ref, wq_ref, k_ref, v_ref, wo_ref, out_ref,
             lhalo_k, lhalo_v, rhalo_k, rhalo_v,
             stage_lo_k, stage_lo_v, stage_hi_k, stage_hi_v,
             send_sems, recv_sems):
        my = lax.axis_index("i")
        left = lax.rem(my + N_DEV - 1, N_DEV)
        right = lax.rem(my + 1, N_DEV)

        stage_lo_k[...] = k_ref[:, :, :, :WIN].astype(jnp.bfloat16)
        stage_lo_v[...] = v_ref[:, :, :, :WIN].astype(jnp.bfloat16)
        stage_hi_k[...] = k_ref[:, :, :, Skv - WIN:].astype(jnp.bfloat16)
        stage_hi_v[...] = v_ref[:, :, :, Skv - WIN:].astype(jnp.bfloat16)

        barrier_sem = pltpu.get_barrier_semaphore()
        for nbr in (left, right):
            pl.semaphore_signal(
                barrier_sem, inc=1,
                device_id=(nbr,), device_id_type=pl.DeviceIdType.MESH,
            )
        pl.semaphore_wait(barrier_sem, 2)

        to_left, to_right = [], []
        for idx, (src, dst) in enumerate(
                [(stage_lo_k, rhalo_k), (stage_lo_v, rhalo_v)]):
            r = pltpu.make_async_remote_copy(
                src_ref=src, dst_ref=dst,
                send_sem=send_sems.at[idx],
                recv_sem=recv_sems.at[idx],
                device_id=(left,), device_id_type=pl.DeviceIdType.MESH,
            )
            r.start()
            to_left.append(r)
        for idx, (src, dst) in enumerate(
                [(stage_hi_k, lhalo_k), (stage_hi_v, lhalo_v)], 2):
            r = pltpu.make_async_remote_copy(
                src_ref=src, dst_ref=dst,
                send_sem=send_sems.at[idx],
                recv_sem=recv_sems.at[idx],
                device_id=(right,), device_id_type=pl.DeviceIdType.MESH,
            )
            r.start()
            to_right.append(r)

        xr = x_ref[...].reshape(B * Sq, HD)
        Qbf = lax.dot(xr, wq_ref[...], preferred_element_type=jnp.float32)

        bi = lax.broadcasted_iota(jnp.int32, (QB, KB), 0)
        bj = lax.broadcasted_iota(jnp.int32, (QB, KB), 1)
        band = jnp.abs(bi - bj + WIN) <= WIN
        neg = jnp.float32(-1e9)

        kt_bf = k_ref[...]
        vt_bf = v_ref[...]

        def attn_block(b, h, qb, halo):
            lo = qb * QB - WIN
            if lo < 0:
                hk, hv = halo
                k = jnp.concatenate([hk[b, h], kt_bf[b, h, :, :KB + lo]], 1)
                v = jnp.concatenate([hv[b, h], vt_bf[b, h, :, :KB + lo]], 1)
                vmask = band & ((bj >= WIN) | (my > 0))
            elif lo + KB > Skv:
                hk, hv = halo
                k = jnp.concatenate([kt_bf[b, h, :, lo:], hk[b, h]], 1)
                v = jnp.concatenate([vt_bf[b, h, :, lo:], hv[b, h]], 1)
                vmask = band & ((bj < KB - WIN) | (my < N_DEV - 1))
            else:
                k = kt_bf[b, h, :, lo:lo + KB]
                v = vt_bf[b, h, :, lo:lo + KB]
                vmask = band
            q = Qbf[b * Sq + qb * QB:b * Sq + (qb + 1) * QB,
                    h * Dh:(h + 1) * Dh]
            s = lax.dot_general(
                q, k, (((1,), (0,)), ((), ())),
                preferred_element_type=jnp.float32,
            ) * 0.125
            p = jnp.exp(jnp.where(vmask, s, neg))
            denom = jnp.sum(p, axis=-1, keepdims=True)
            ctx = lax.dot_general(
                p, v, (((1,), (1,)), ((), ())),
                preferred_element_type=jnp.float32)
            return ctx / denom

        wo_bf = wo_ref[...]

        def out_rows(b, qb, halo):
            cb = jnp.concatenate(
                [attn_block(b, h, qb, halo) for h in range(Hq)], axis=1)
            out_ref[b, qb * QB:(qb + 1) * QB, :] = lax.dot(
                cb, wo_bf, preferred_element_type=jnp.float32)

        for b in range(B):
            for qb in (1, 2):
                out_rows(b, qb, None)

        for r in to_right:
            r.wait_recv()
        lk = lhalo_k[...].astype(jnp.float32)
        lv = lhalo_v[...].astype(jnp.float32)
        for b in range(B):
            out_rows(b, 0, (lk, lv))

        for r in to_left:
            r.wait_recv()
        rk = rhalo_k[...].astype(jnp.float32)
        rv = rhalo_v[...].astype(jnp.float32)
        for b in range(B):
            out_rows(b, NQB - 1, (rk, rv))

        for r in to_left + to_right:
            r.wait_send()

    halo = pltpu.VMEM((B, Hq, Dh, WIN), jnp.bfloat16)
    return pl.pallas_call(
        body,
        out_shape=jax.ShapeDtypeStruct((B, Sq, HD), jnp.float32),
        in_specs=[pl.BlockSpec(memory_space=pltpu.VMEM)] * 5,
        out_specs=pl.BlockSpec(memory_space=pltpu.VMEM),
        scratch_shapes=[
            halo, halo, halo, halo,
            halo, halo, halo, halo,
            pltpu.SemaphoreType.DMA((4,)),
            pltpu.SemaphoreType.DMA((4,)),
        ],
        compiler_params=pltpu.CompilerParams(collective_id=0),
    )(x, Wq, Kt, Vt, Wo)
